# baseline (speedup 1.0000x reference)
import jax
import jax.numpy as jnp
import numpy as np
from jax.experimental import pallas as pl
from jax.experimental.pallas import tpu as pltpu

_BF = jnp.bfloat16
_F32 = jnp.float32


def _interp_mat(out_size, in_size):
    if out_size == 1:
        m = np.zeros((1, in_size), np.float32)
        m[0, 0] = 1.0
        return m
    pos = np.arange(out_size, dtype=np.float64) * (
        float(in_size - 1) / float(out_size - 1))
    i0 = np.clip(np.floor(pos).astype(np.int64), 0, in_size - 1)
    i1 = np.minimum(i0 + 1, in_size - 1)
    fr = (pos - i0).astype(np.float32)
    m = np.zeros((out_size, in_size), np.float32)
    rows = np.arange(out_size)
    m[rows, i0] += 1.0 - fr
    m[rows, i1] += fr
    return m


def _pool_mat(in_size):
    out_size = (in_size - 1) // 2 + 1
    m = np.zeros((out_size, in_size), np.float32)
    for o in range(out_size):
        for k in (2 * o - 1, 2 * o, 2 * o + 1):
            if 0 <= k < in_size:
                m[o, k] += 1.0 / 3.0
    return m


def _down_mat(out_size, in_size):
    p = _pool_mat(in_size)
    return _interp_mat(out_size, p.shape[0]) @ p


def _swish(x):
    return x * jax.nn.sigmoid(x)


def _conv(w, xb):
    return jnp.einsum("dc,chw->dhw", w, xb, preferred_element_type=_F32)


def _hmix(r, x):
    c = x.shape[0]
    rb = jnp.broadcast_to(r[None], (c,) + r.shape)
    return jnp.einsum("coh,chw->cow", rb, x, preferred_element_type=_F32)


def _wmix(x, rt):
    c, h, wi = x.shape
    y = jnp.dot(x.reshape(c * h, wi), rt, preferred_element_type=_F32)
    return y.astype(_BF).reshape(c, h, rt.shape[1])


def _k67_body(p6_ref, p7_ref, w67_ref, b_ref, wn_ref, mid_ref, y_ref):
    xc = jnp.concatenate([p6_ref[0].astype(_BF), p7_ref[0].astype(_BF)], axis=0)
    acc = jnp.dot(w67_ref[...], xc, preferred_element_type=_F32)
    midb = _swish(acc + b_ref[...].astype(_F32)).astype(_BF)
    mid_ref[0] = midb
    y_ref[0] = jnp.dot(wn_ref[...], midb, preferred_element_type=_F32).astype(_BF)


def _k45_body(p4_ref, p5_ref, y6_ref, w45_ref, b45_ref, wb34_ref,
              wa56_ref, b56_ref, kup_ref, p4mid_ref, y4_ref, p5mid_ref):
    x5 = p5_ref[0].astype(_BF)
    xc = jnp.concatenate([p4_ref[0].astype(_BF), x5], axis=0)
    acc4 = jnp.dot(w45_ref[...], xc, preferred_element_type=_F32)
    mid4 = _swish(acc4 + b45_ref[...].astype(_F32)).astype(_BF)
    p4mid_ref[0] = mid4
    y4_ref[0] = jnp.dot(wb34_ref[...], mid4,
                        preferred_element_type=_F32).astype(_BF)
    up = jnp.dot(y6_ref[0], kup_ref[...], preferred_element_type=_F32)
    acc5 = jnp.dot(wa56_ref[...], x5, preferred_element_type=_F32) + up
    p5mid_ref[0] = _swish(acc5 + b56_ref[...].astype(_F32)).astype(_BF)


def _k3_body(p3_ref, y4_ref, rh43_ref, rw43t_ref, wa34_ref, b34_ref,
             dw34t_ref, out_ref, p3w_ref):
    up = _hmix(rh43_ref[...], _wmix(y4_ref[0], rw43t_ref[...]))
    acc = _conv(wa34_ref[...], p3_ref[0].astype(_BF)) + up
    p3o = _swish(acc + b34_ref[...].astype(_F32))
    out_ref[0] = p3o
    p3w_ref[0] = _wmix(p3o.astype(_BF), dw34t_ref[...])


def _k4_body(p3w_ref, p4_ref, p4mid_ref, dh34_ref, w4_ref, b4_ref,
             out_ref):
    dwn = _hmix(dh34_ref[...], p3w_ref[0])
    s = (p4_ref[0].astype(_F32) + p4mid_ref[0].astype(_F32) + dwn).astype(_BF)
    out_ref[0] = _swish(_conv(w4_ref[...], s) + b4_ref[...].astype(_F32))


def _k5_body(p4o_ref, p5_ref, p5mid_ref, p6_ref, p6mid_ref, p7_ref,
             w5_ref, b5_ref, dh56_ref, dw56t_ref, w6_ref, b6_ref,
             w7_ref, b7_ref, p5o_ref, p6o_ref, p7o_ref):
    s5 = (p5_ref[0].astype(_F32) + p5mid_ref[0].astype(_F32)
          + p4o_ref[0].astype(_F32)).astype(_BF)
    p5o = _swish(_conv(w5_ref[...], s5) + b5_ref[...].astype(_F32))
    p5o_ref[0] = p5o
    dwn = _hmix(dh56_ref[...], _wmix(p5o.astype(_BF), dw56t_ref[...]))
    s6 = (p6_ref[0].astype(_F32) + p6mid_ref[0].astype(_F32) + dwn).astype(_BF)
    p6o = _swish(_conv(w6_ref[...], s6) + b6_ref[...].astype(_F32))
    p6o_ref[0] = p6o
    s7 = (p7_ref[0].astype(_F32) + p6o).astype(_BF)
    p7o_ref[0] = _swish(_conv(w7_ref[...], s7) + b7_ref[...].astype(_F32))


def _cparams(grid_rank):
    return pltpu.CompilerParams(
        dimension_semantics=("parallel",) * grid_rank,
        vmem_limit_bytes=56 << 20,
    )


def _full(shape):
    return pl.BlockSpec(shape, lambda b: (b,) + (0,) * (len(shape) - 1))


def _const(shape):
    return pl.BlockSpec(shape, lambda *_: (0,) * len(shape))


def kernel(p3, p4, p5, p6, p7,
           convp67_w, convp67_b, convp56_w, convp56_b,
           convp45_w, convp45_b, convp34_w, convp34_b,
           out4_w, out4_b, out4_gamma, out4_beta, out4_mean, out4_var,
           out5_w, out5_b, out5_gamma, out5_beta, out5_mean, out5_var,
           out6_w, out6_b, out6_gamma, out6_beta, out6_mean, out6_var,
           out7_w, out7_b, out7_gamma, out7_beta, out7_mean, out7_var):
    B, C, H3, W3 = p3.shape
    H4, W4 = p4.shape[2:]
    H6, W6 = p6.shape[2:]
    eps = 1e-5

    fb = lambda a: a.astype(_BF)
    col = lambda v: v.reshape(C, 1, 1)

    w67 = fb(convp67_w)
    w45 = fb(convp45_w)
    wa56, wb56 = fb(convp56_w[:, :C]), fb(convp56_w[:, C:])
    wb34 = fb(convp34_w[:, C:])
    wa34 = fb(convp34_w[:, :C])
    col2 = lambda v: v.reshape(C, 1)
    b67c, b56c, b45c = col2(convp67_b), col2(convp56_b), col2(convp45_b)
    b34 = col(convp34_b)

    def bn_fold(w, b, gamma, beta, mean, var):
        s = gamma * jax.lax.rsqrt(var + eps)
        return fb(w * s[:, None]), col((b - mean) * s + beta)

    w4e, b4e = bn_fold(out4_w, out4_b, out4_gamma, out4_beta, out4_mean, out4_var)
    w5e, b5e = bn_fold(out5_w, out5_b, out5_gamma, out5_beta, out5_mean, out5_var)
    w6e, b6e = bn_fold(out6_w, out6_b, out6_gamma, out6_beta, out6_mean, out6_var)
    w7e, b7e = bn_fold(out7_w, out7_b, out7_gamma, out7_beta, out7_mean, out7_var)

    mat = lambda m: jnp.asarray(m, _BF)
    rh43, rw43t = mat(_interp_mat(H3, H4)), mat(_interp_mat(W3, W4).T)
    dh34, dw34t = mat(_down_mat(H4, H3)), mat(_down_mat(W4, W3).T)
    dh56, dw56t = mat(_down_mat(H6, H4)), mat(_down_mat(W6, W4).T)
    kup65t = mat(np.kron(_interp_mat(H4, H6), _interp_mat(W4, W6)).T)

    cc = _const

    M6, M4 = H6 * W6, H4 * W4
    p6f = p6.reshape(B, C, M6)
    p7f = p7.reshape(B, C, M6)
    p6_mid, y6 = pl.pallas_call(
        _k67_body,
        out_shape=(jax.ShapeDtypeStruct((B, C, M6), _BF),
                   jax.ShapeDtypeStruct((B, C, M6), _BF)),
        grid=(B,),
        in_specs=[_full((1, C, M6)), _full((1, C, M6)),
                  cc((C, 2 * C)), cc((C, 1)), cc((C, C))],
        out_specs=[_full((1, C, M6)), _full((1, C, M6))],
        compiler_params=_cparams(1),
    )(p6f, p7f, w67, b67c, wb56)

    th4 = H4 // 2
    p4_mid, y4, p5_mid = pl.pallas_call(
        _k45_body,
        out_shape=(jax.ShapeDtypeStruct((B, C, M4), _BF),
                   jax.ShapeDtypeStruct((B, C, M4), _BF),
                   jax.ShapeDtypeStruct((B, C, M4), _BF)),
        grid=(B,),
        in_specs=[_full((1, C, M4)), _full((1, C, M4)), _full((1, C, M6)),
                  cc((C, 2 * C)), cc((C, 1)), cc((C, C)),
                  cc((C, C)), cc((C, 1)), cc((M6, M4))],
        out_specs=[_full((1, C, M4))] * 3,
        compiler_params=_cparams(1),
    )(p4.reshape(B, C, M4), p5.reshape(B, C, M4), y6,
      w45, b45c, wb34, wa56, b56c, kup65t)

    th = H3 // 2
    p3_out, p3w = pl.pallas_call(
        _k3_body,
        out_shape=(jax.ShapeDtypeStruct((B, C, H3, W3), _F32),
                   jax.ShapeDtypeStruct((B, C, H3, W4), _BF)),
        grid=(B, 2),
        in_specs=[
            pl.BlockSpec((1, C, th, W3), lambda b, i: (b, 0, i, 0)),
            pl.BlockSpec((1, C, H4, W4), lambda b, i: (b, 0, 0, 0)),
            pl.BlockSpec((th, H4), lambda b, i: (i, 0)),
            pl.BlockSpec((W4, W3), lambda b, i: (0, 0)),
            pl.BlockSpec((C, C), lambda b, i: (0, 0)),
            pl.BlockSpec((C, 1, 1), lambda b, i: (0, 0, 0)),
            pl.BlockSpec((W3, W4), lambda b, i: (0, 0)),
        ],
        out_specs=[pl.BlockSpec((1, C, th, W3), lambda b, i: (b, 0, i, 0)),
                   pl.BlockSpec((1, C, th, W4), lambda b, i: (b, 0, i, 0))],
        compiler_params=_cparams(2),
    )(p3, y4.reshape(B, C, H4, W4), rh43, rw43t, wa34, b34, dw34t)

    p4_out = pl.pallas_call(
        _k4_body,
        out_shape=jax.ShapeDtypeStruct((B, C, H4, W4), _F32),
        grid=(B, 2),
        in_specs=[
            pl.BlockSpec((1, C, H3, W4), lambda b, i: (b, 0, 0, 0)),
            pl.BlockSpec((1, C, th4, W4), lambda b, i: (b, 0, i, 0)),
            pl.BlockSpec((1, C, th4, W4), lambda b, i: (b, 0, i, 0)),
            pl.BlockSpec((th4, H3), lambda b, i: (i, 0)),
            pl.BlockSpec((C, C), lambda b, i: (0, 0)),
            pl.BlockSpec((C, 1, 1), lambda b, i: (0, 0, 0)),
        ],
        out_specs=pl.BlockSpec((1, C, th4, W4), lambda b, i: (b, 0, i, 0)),
        compiler_params=_cparams(2),
    )(p3w, p4, p4_mid.reshape(B, C, H4, W4), dh34, w4e, b4e)

    p5_out, p6_out, p7_out = pl.pallas_call(
        _k5_body,
        out_shape=(jax.ShapeDtypeStruct((B, C, H4, W4), _F32),
                   jax.ShapeDtypeStruct((B, C, H6, W6), _F32),
                   jax.ShapeDtypeStruct((B, C, H6, W6), _F32)),
        grid=(B,),
        in_specs=[_full((1, C, H4, W4)), _full((1, C, H4, W4)),
                  _full((1, C, H4, W4)), _full((1, C, H6, W6)),
                  _full((1, C, H6, W6)), _full((1, C, H6, W6)),
                  cc((C, C)), cc((C, 1, 1)),
                  cc((H6, H4)), cc((W4, W6)),
                  cc((C, C)), cc((C, 1, 1)),
                  cc((C, C)), cc((C, 1, 1))],
        out_specs=[_full((1, C, H4, W4)), _full((1, C, H6, W6)),
                   _full((1, C, H6, W6))],
        compiler_params=_cparams(1),
    )(p4_out, p5, p5_mid.reshape(B, C, H4, W4), p6,
      p6_mid.reshape(B, C, H6, W6), p7,
      w5e, b5e, dh56, dw56t, w6e, b6e, w7e, b7e)

    return [p3_out, p4_out, p5_out, p6_out, p7_out]


# speedup vs baseline: 1.0304x; 1.0304x over previous
import jax
import jax.numpy as jnp
import numpy as np
from jax.experimental import pallas as pl
from jax.experimental.pallas import tpu as pltpu

_BF = jnp.bfloat16
_F32 = jnp.float32


def _interp_mat(out_size, in_size):
    if out_size == 1:
        m = np.zeros((1, in_size), np.float32)
        m[0, 0] = 1.0
        return m
    pos = np.arange(out_size, dtype=np.float64) * (
        float(in_size - 1) / float(out_size - 1))
    i0 = np.clip(np.floor(pos).astype(np.int64), 0, in_size - 1)
    i1 = np.minimum(i0 + 1, in_size - 1)
    fr = (pos - i0).astype(np.float32)
    m = np.zeros((out_size, in_size), np.float32)
    rows = np.arange(out_size)
    m[rows, i0] += 1.0 - fr
    m[rows, i1] += fr
    return m


def _pool_mat(in_size):
    out_size = (in_size - 1) // 2 + 1
    m = np.zeros((out_size, in_size), np.float32)
    for o in range(out_size):
        for k in (2 * o - 1, 2 * o, 2 * o + 1):
            if 0 <= k < in_size:
                m[o, k] += 1.0 / 3.0
    return m


def _down_mat(out_size, in_size):
    p = _pool_mat(in_size)
    return _interp_mat(out_size, p.shape[0]) @ p


def _swish(x):
    return x * jax.nn.sigmoid(x)


def _conv(w, xb):
    return jnp.einsum("dc,chw->dhw", w, xb, preferred_element_type=_F32)


def _hmix(r, x):
    c = x.shape[0]
    rb = jnp.broadcast_to(r[None], (c,) + r.shape)
    return jnp.einsum("coh,chw->cow", rb, x, preferred_element_type=_F32)


def _wmix(x, rt):
    c, h, wi = x.shape
    y = jnp.dot(x.reshape(c * h, wi), rt, preferred_element_type=_F32)
    return y.astype(_BF).reshape(c, h, rt.shape[1])


def _unflatten(xf, h, w):
    return jnp.stack([xf[:, i * w:(i + 1) * w] for i in range(h)], axis=1)


def _k67_body(p6_ref, p7_ref, w67_ref, b_ref, wn_ref, mid_ref, y_ref):
    xc = jnp.concatenate([p6_ref[0].astype(_BF), p7_ref[0].astype(_BF)], axis=0)
    acc = jnp.dot(w67_ref[...], xc, preferred_element_type=_F32)
    midb = _swish(acc + b_ref[...].astype(_F32)).astype(_BF)
    mid_ref[0] = midb
    y_ref[0] = jnp.dot(wn_ref[...], midb, preferred_element_type=_F32).astype(_BF)


def _k45_body(p4_ref, p5_ref, y6_ref, w45_ref, b45_ref, wb34_ref,
              wa56_ref, b56_ref, kup_ref, p4mid_ref, y4_ref, p5mid_ref):
    x5 = p5_ref[0].astype(_BF)
    xc = jnp.concatenate([p4_ref[0].astype(_BF), x5], axis=0)
    acc4 = jnp.dot(w45_ref[...], xc, preferred_element_type=_F32)
    mid4 = _swish(acc4 + b45_ref[...].astype(_F32)).astype(_BF)
    p4mid_ref[0] = mid4
    y4_ref[0] = jnp.dot(wb34_ref[...], mid4,
                        preferred_element_type=_F32).astype(_BF)
    up = jnp.dot(y6_ref[0], kup_ref[...], preferred_element_type=_F32)
    acc5 = jnp.dot(wa56_ref[...], x5, preferred_element_type=_F32) + up
    p5mid_ref[0] = _swish(acc5 + b56_ref[...].astype(_F32)).astype(_BF)


def _k3_body(p3_ref, y4_ref, rh43_ref, rw43t_ref, wa34_ref, b34_ref,
             dw34t_ref, out_ref, p3w_ref):
    c, m4 = y4_ref.shape[1:]
    h4 = rh43_ref.shape[1]
    y43 = _unflatten(y4_ref[0], h4, m4 // h4)
    up = _hmix(rh43_ref[...], _wmix(y43, rw43t_ref[...]))
    acc = _conv(wa34_ref[...], p3_ref[0].astype(_BF)) + up
    p3o = _swish(acc + b34_ref[...].astype(_F32))
    out_ref[0] = p3o
    p3w_ref[0] = _wmix(p3o.astype(_BF), dw34t_ref[...])


def _k4_body(p3w_ref, p4_ref, p4mid_ref, dh34_ref, w4_ref, b4_ref,
             out_ref):
    th, w4 = out_ref.shape[2:]
    dwn = _hmix(dh34_ref[...], p3w_ref[0])
    p4m = _unflatten(p4mid_ref[0], th, w4)
    s = (p4_ref[0].astype(_F32) + p4m.astype(_F32) + dwn).astype(_BF)
    out_ref[0] = _swish(_conv(w4_ref[...], s) + b4_ref[...].astype(_F32))


def _k5_body(p4o_ref, p5_ref, p5mid_ref, p6_ref, p6mid_ref, p7_ref,
             w5_ref, b5_ref, dh56_ref, dw56t_ref, w6_ref, b6_ref,
             w7_ref, b7_ref, p5o_ref, p6o_ref, p7o_ref):
    h4, w4 = p5_ref.shape[2:]
    h6, w6 = p6_ref.shape[2:]
    p5m = _unflatten(p5mid_ref[0], h4, w4)
    s5 = (p5_ref[0].astype(_F32) + p5m.astype(_F32)
          + p4o_ref[0].astype(_F32)).astype(_BF)
    p5o = _swish(_conv(w5_ref[...], s5) + b5_ref[...].astype(_F32))
    p5o_ref[0] = p5o
    dwn = _hmix(dh56_ref[...], _wmix(p5o.astype(_BF), dw56t_ref[...]))
    p6m = _unflatten(p6mid_ref[0], h6, w6)
    s6 = (p6_ref[0].astype(_F32) + p6m.astype(_F32) + dwn).astype(_BF)
    p6o = _swish(_conv(w6_ref[...], s6) + b6_ref[...].astype(_F32))
    p6o_ref[0] = p6o
    s7 = (p7_ref[0].astype(_F32) + p6o).astype(_BF)
    p7o_ref[0] = _swish(_conv(w7_ref[...], s7) + b7_ref[...].astype(_F32))


def _cparams(grid_rank):
    return pltpu.CompilerParams(
        dimension_semantics=("parallel",) * grid_rank,
        vmem_limit_bytes=56 << 20,
    )


def _full(shape):
    return pl.BlockSpec(shape, lambda b: (b,) + (0,) * (len(shape) - 1))


def _const(shape):
    return pl.BlockSpec(shape, lambda *_: (0,) * len(shape))


def kernel(p3, p4, p5, p6, p7,
           convp67_w, convp67_b, convp56_w, convp56_b,
           convp45_w, convp45_b, convp34_w, convp34_b,
           out4_w, out4_b, out4_gamma, out4_beta, out4_mean, out4_var,
           out5_w, out5_b, out5_gamma, out5_beta, out5_mean, out5_var,
           out6_w, out6_b, out6_gamma, out6_beta, out6_mean, out6_var,
           out7_w, out7_b, out7_gamma, out7_beta, out7_mean, out7_var):
    B, C, H3, W3 = p3.shape
    H4, W4 = p4.shape[2:]
    H6, W6 = p6.shape[2:]
    eps = 1e-5

    fb = lambda a: a.astype(_BF)
    col = lambda v: v.reshape(C, 1, 1)

    w67 = fb(convp67_w)
    w45 = fb(convp45_w)
    wa56, wb56 = fb(convp56_w[:, :C]), fb(convp56_w[:, C:])
    wb34 = fb(convp34_w[:, C:])
    wa34 = fb(convp34_w[:, :C])
    col2 = lambda v: v.reshape(C, 1)
    b67c, b56c, b45c = col2(convp67_b), col2(convp56_b), col2(convp45_b)
    b34 = col(convp34_b)

    def bn_fold(w, b, gamma, beta, mean, var):
        s = gamma * jax.lax.rsqrt(var + eps)
        return fb(w * s[:, None]), col((b - mean) * s + beta)

    w4e, b4e = bn_fold(out4_w, out4_b, out4_gamma, out4_beta, out4_mean, out4_var)
    w5e, b5e = bn_fold(out5_w, out5_b, out5_gamma, out5_beta, out5_mean, out5_var)
    w6e, b6e = bn_fold(out6_w, out6_b, out6_gamma, out6_beta, out6_mean, out6_var)
    w7e, b7e = bn_fold(out7_w, out7_b, out7_gamma, out7_beta, out7_mean, out7_var)

    mat = lambda m: jnp.asarray(m, _BF)
    rh43, rw43t = mat(_interp_mat(H3, H4)), mat(_interp_mat(W3, W4).T)
    dh34, dw34t = mat(_down_mat(H4, H3)), mat(_down_mat(W4, W3).T)
    dh56, dw56t = mat(_down_mat(H6, H4)), mat(_down_mat(W6, W4).T)
    kup65t = mat(np.kron(_interp_mat(H4, H6), _interp_mat(W4, W6)).T)

    cc = _const

    M6, M4 = H6 * W6, H4 * W4
    p6f = p6.reshape(B, C, M6)
    p7f = p7.reshape(B, C, M6)
    p6_mid, y6 = pl.pallas_call(
        _k67_body,
        out_shape=(jax.ShapeDtypeStruct((B, C, M6), _BF),
                   jax.ShapeDtypeStruct((B, C, M6), _BF)),
        grid=(B,),
        in_specs=[_full((1, C, M6)), _full((1, C, M6)),
                  cc((C, 2 * C)), cc((C, 1)), cc((C, C))],
        out_specs=[_full((1, C, M6)), _full((1, C, M6))],
        compiler_params=_cparams(1),
    )(p6f, p7f, w67, b67c, wb56)

    th4 = H4 // 2
    p4_mid, y4, p5_mid = pl.pallas_call(
        _k45_body,
        out_shape=(jax.ShapeDtypeStruct((B, C, M4), _BF),
                   jax.ShapeDtypeStruct((B, C, M4), _BF),
                   jax.ShapeDtypeStruct((B, C, M4), _BF)),
        grid=(B,),
        in_specs=[_full((1, C, M4)), _full((1, C, M4)), _full((1, C, M6)),
                  cc((C, 2 * C)), cc((C, 1)), cc((C, C)),
                  cc((C, C)), cc((C, 1)), cc((M6, M4))],
        out_specs=[_full((1, C, M4))] * 3,
        compiler_params=_cparams(1),
    )(p4.reshape(B, C, M4), p5.reshape(B, C, M4), y6,
      w45, b45c, wb34, wa56, b56c, kup65t)

    th = H3 // 2
    p3_out, p3w = pl.pallas_call(
        _k3_body,
        out_shape=(jax.ShapeDtypeStruct((B, C, H3, W3), _F32),
                   jax.ShapeDtypeStruct((B, C, H3, W4), _BF)),
        grid=(B, 2),
        in_specs=[
            pl.BlockSpec((1, C, th, W3), lambda b, i: (b, 0, i, 0)),
            pl.BlockSpec((1, C, M4), lambda b, i: (b, 0, 0)),
            pl.BlockSpec((th, H4), lambda b, i: (i, 0)),
            pl.BlockSpec((W4, W3), lambda b, i: (0, 0)),
            pl.BlockSpec((C, C), lambda b, i: (0, 0)),
            pl.BlockSpec((C, 1, 1), lambda b, i: (0, 0, 0)),
            pl.BlockSpec((W3, W4), lambda b, i: (0, 0)),
        ],
        out_specs=[pl.BlockSpec((1, C, th, W3), lambda b, i: (b, 0, i, 0)),
                   pl.BlockSpec((1, C, th, W4), lambda b, i: (b, 0, i, 0))],
        compiler_params=_cparams(2),
    )(p3, y4, rh43, rw43t, wa34, b34, dw34t)

    p4_out = pl.pallas_call(
        _k4_body,
        out_shape=jax.ShapeDtypeStruct((B, C, H4, W4), _F32),
        grid=(B, 2),
        in_specs=[
            pl.BlockSpec((1, C, H3, W4), lambda b, i: (b, 0, 0, 0)),
            pl.BlockSpec((1, C, th4, W4), lambda b, i: (b, 0, i, 0)),
            pl.BlockSpec((1, C, th4 * W4), lambda b, i: (b, 0, i)),
            pl.BlockSpec((th4, H3), lambda b, i: (i, 0)),
            pl.BlockSpec((C, C), lambda b, i: (0, 0)),
            pl.BlockSpec((C, 1, 1), lambda b, i: (0, 0, 0)),
        ],
        out_specs=pl.BlockSpec((1, C, th4, W4), lambda b, i: (b, 0, i, 0)),
        compiler_params=_cparams(2),
    )(p3w, p4, p4_mid, dh34, w4e, b4e)

    p5_out, p6_out, p7_out = pl.pallas_call(
        _k5_body,
        out_shape=(jax.ShapeDtypeStruct((B, C, H4, W4), _F32),
                   jax.ShapeDtypeStruct((B, C, H6, W6), _F32),
                   jax.ShapeDtypeStruct((B, C, H6, W6), _F32)),
        grid=(B,),
        in_specs=[_full((1, C, H4, W4)), _full((1, C, H4, W4)),
                  _full((1, C, M4)), _full((1, C, H6, W6)),
                  _full((1, C, M6)), _full((1, C, H6, W6)),
                  cc((C, C)), cc((C, 1, 1)),
                  cc((H6, H4)), cc((W4, W6)),
                  cc((C, C)), cc((C, 1, 1)),
                  cc((C, C)), cc((C, 1, 1))],
        out_specs=[_full((1, C, H4, W4)), _full((1, C, H6, W6)),
                   _full((1, C, H6, W6))],
        compiler_params=_cparams(1),
    )(p4_out, p5, p5_mid, p6, p6_mid, p7,
      w5e, b5e, dh56, dw56t, w6e, b6e, w7e, b7e)

    return [p3_out, p4_out, p5_out, p6_out, p7_out]


# speedup vs baseline: 1.1695x; 1.1350x over previous
import jax
import jax.numpy as jnp
import numpy as np
from jax.experimental import pallas as pl
from jax.experimental.pallas import tpu as pltpu

_BF = jnp.bfloat16
_F32 = jnp.float32


def _interp_mat(out_size, in_size):
    if out_size == 1:
        m = np.zeros((1, in_size), np.float32)
        m[0, 0] = 1.0
        return m
    pos = np.arange(out_size, dtype=np.float64) * (
        float(in_size - 1) / float(out_size - 1))
    i0 = np.clip(np.floor(pos).astype(np.int64), 0, in_size - 1)
    i1 = np.minimum(i0 + 1, in_size - 1)
    fr = (pos - i0).astype(np.float32)
    m = np.zeros((out_size, in_size), np.float32)
    rows = np.arange(out_size)
    m[rows, i0] += 1.0 - fr
    m[rows, i1] += fr
    return m


def _pool_mat(in_size):
    out_size = (in_size - 1) // 2 + 1
    m = np.zeros((out_size, in_size), np.float32)
    for o in range(out_size):
        for k in (2 * o - 1, 2 * o, 2 * o + 1):
            if 0 <= k < in_size:
                m[o, k] += 1.0 / 3.0
    return m


def _down_mat(out_size, in_size):
    p = _pool_mat(in_size)
    return _interp_mat(out_size, p.shape[0]) @ p


def _swish(x):
    return x * jax.nn.sigmoid(x)


def _conv(w, xb):
    return jnp.einsum("dc,chw->dhw", w, xb, preferred_element_type=_F32)


def _hmix(r, x):
    c = x.shape[0]
    rb = jnp.broadcast_to(r[None], (c,) + r.shape)
    return jnp.einsum("coh,chw->cow", rb, x, preferred_element_type=_F32)


def _wmix(x, rt):
    c, h, wi = x.shape
    y = jnp.dot(x.reshape(c * h, wi), rt, preferred_element_type=_F32)
    return y.astype(_BF).reshape(c, h, rt.shape[1])


def _k67_body(p6_ref, p7_ref, w67_ref, b_ref, wn_ref, mid_ref, y_ref):
    xc = jnp.concatenate([p6_ref[0].astype(_BF), p7_ref[0].astype(_BF)], axis=0)
    midb = _swish(_conv(w67_ref[...], xc) + b_ref[...].astype(_F32)).astype(_BF)
    mid_ref[0] = midb
    y_ref[0] = _conv(wn_ref[...], midb).astype(_BF)


def _k45_body(p4_ref, p5_ref, y6_ref, w45_ref, b45_ref, wb34_ref,
              wa56_ref, b56_ref, rh65_ref, rw65t_ref,
              p4mid_ref, y4_ref, p5mid_ref):
    x5 = p5_ref[0].astype(_BF)
    xc = jnp.concatenate([p4_ref[0].astype(_BF), x5], axis=0)
    mid4 = _swish(_conv(w45_ref[...], xc) + b45_ref[...].astype(_F32)).astype(_BF)
    p4mid_ref[0] = mid4
    y4_ref[0] = _conv(wb34_ref[...], mid4).astype(_BF)
    up = _hmix(rh65_ref[...], _wmix(y6_ref[0], rw65t_ref[...]))
    acc5 = _conv(wa56_ref[...], x5) + up + b56_ref[...].astype(_F32)
    p5mid_ref[0] = _swish(acc5).astype(_BF)


def _k3_body(p3_ref, y4_ref, rh43_ref, rw43t_ref, wa34_ref, b34_ref,
             dw34t_ref, out_ref, p3w_ref):
    up = _hmix(rh43_ref[...], _wmix(y4_ref[0], rw43t_ref[...]))
    acc = _conv(wa34_ref[...], p3_ref[0].astype(_BF)) + up
    p3o = _swish(acc + b34_ref[...].astype(_F32))
    out_ref[0] = p3o
    p3w_ref[0] = _wmix(p3o.astype(_BF), dw34t_ref[...])


def _k4_body(p3w_ref, p4_ref, p4mid_ref, dh34_ref, w4_ref, b4_ref,
             out_ref):
    dwn = _hmix(dh34_ref[...], p3w_ref[0])
    s = (p4_ref[0].astype(_F32) + p4mid_ref[0].astype(_F32) + dwn).astype(_BF)
    out_ref[0] = _swish(_conv(w4_ref[...], s) + b4_ref[...].astype(_F32))


def _k5_body(p4o_ref, p5_ref, p5mid_ref, p6_ref, p6mid_ref, p7_ref,
             w5_ref, b5_ref, dh56_ref, dw56t_ref, w6_ref, b6_ref,
             w7_ref, b7_ref, p5o_ref, p6o_ref, p7o_ref):
    s5 = (p5_ref[0].astype(_F32) + p5mid_ref[0].astype(_F32)
          + p4o_ref[0].astype(_F32)).astype(_BF)
    p5o = _swish(_conv(w5_ref[...], s5) + b5_ref[...].astype(_F32))
    p5o_ref[0] = p5o
    dwn = _hmix(dh56_ref[...], _wmix(p5o.astype(_BF), dw56t_ref[...]))
    s6 = (p6_ref[0].astype(_F32) + p6mid_ref[0].astype(_F32) + dwn).astype(_BF)
    p6o = _swish(_conv(w6_ref[...], s6) + b6_ref[...].astype(_F32))
    p6o_ref[0] = p6o
    s7 = (p7_ref[0].astype(_F32) + p6o).astype(_BF)
    p7o_ref[0] = _swish(_conv(w7_ref[...], s7) + b7_ref[...].astype(_F32))


def _cparams(grid_rank):
    return pltpu.CompilerParams(
        dimension_semantics=("parallel",) * grid_rank,
        vmem_limit_bytes=56 << 20,
    )


def _full(shape):
    return pl.BlockSpec(shape, lambda b: (b,) + (0,) * (len(shape) - 1))


def _const(shape):
    return pl.BlockSpec(shape, lambda *_: (0,) * len(shape))


def kernel(p3, p4, p5, p6, p7,
           convp67_w, convp67_b, convp56_w, convp56_b,
           convp45_w, convp45_b, convp34_w, convp34_b,
           out4_w, out4_b, out4_gamma, out4_beta, out4_mean, out4_var,
           out5_w, out5_b, out5_gamma, out5_beta, out5_mean, out5_var,
           out6_w, out6_b, out6_gamma, out6_beta, out6_mean, out6_var,
           out7_w, out7_b, out7_gamma, out7_beta, out7_mean, out7_var):
    B, C, H3, W3 = p3.shape
    H4, W4 = p4.shape[2:]
    H6, W6 = p6.shape[2:]
    eps = 1e-5

    fb = lambda a: a.astype(_BF)
    col = lambda v: v.reshape(C, 1, 1)

    w67 = fb(convp67_w)
    w45 = fb(convp45_w)
    wa56, wb56 = fb(convp56_w[:, :C]), fb(convp56_w[:, C:])
    wb34 = fb(convp34_w[:, C:])
    wa34 = fb(convp34_w[:, :C])
    b67, b56, b45, b34 = (col(convp67_b), col(convp56_b),
                          col(convp45_b), col(convp34_b))

    def bn_fold(w, b, gamma, beta, mean, var):
        s = gamma * jax.lax.rsqrt(var + eps)
        return fb(w * s[:, None]), col((b - mean) * s + beta)

    w4e, b4e = bn_fold(out4_w, out4_b, out4_gamma, out4_beta, out4_mean, out4_var)
    w5e, b5e = bn_fold(out5_w, out5_b, out5_gamma, out5_beta, out5_mean, out5_var)
    w6e, b6e = bn_fold(out6_w, out6_b, out6_gamma, out6_beta, out6_mean, out6_var)
    w7e, b7e = bn_fold(out7_w, out7_b, out7_gamma, out7_beta, out7_mean, out7_var)

    mat = lambda m: jnp.asarray(m, _BF)
    rh65, rw65t = mat(_interp_mat(H4, H6)), mat(_interp_mat(W4, W6).T)
    rh43, rw43t = mat(_interp_mat(H3, H4)), mat(_interp_mat(W3, W4).T)
    dh34, dw34t = mat(_down_mat(H4, H3)), mat(_down_mat(W4, W3).T)
    dh56, dw56t = mat(_down_mat(H6, H4)), mat(_down_mat(W6, W4).T)

    cc = _const

    p6_mid, y6 = pl.pallas_call(
        _k67_body,
        out_shape=(jax.ShapeDtypeStruct((B, C, H6, W6), _BF),
                   jax.ShapeDtypeStruct((B, C, H6, W6), _BF)),
        grid=(B,),
        in_specs=[_full((1, C, H6, W6)), _full((1, C, H6, W6)),
                  cc((C, 2 * C)), cc((C, 1, 1)), cc((C, C))],
        out_specs=[_full((1, C, H6, W6)), _full((1, C, H6, W6))],
        compiler_params=_cparams(1),
    )(p6, p7, w67, b67, wb56)

    th4 = H4 // 2
    p4_mid, y4, p5_mid = pl.pallas_call(
        _k45_body,
        out_shape=(jax.ShapeDtypeStruct((B, C, H4, W4), _BF),
                   jax.ShapeDtypeStruct((B, C, H4, W4), _BF),
                   jax.ShapeDtypeStruct((B, C, H4, W4), _BF)),
        grid=(B, 2),
        in_specs=[
            pl.BlockSpec((1, C, th4, W4), lambda b, i: (b, 0, i, 0)),
            pl.BlockSpec((1, C, th4, W4), lambda b, i: (b, 0, i, 0)),
            pl.BlockSpec((1, C, H6, W6), lambda b, i: (b, 0, 0, 0)),
            pl.BlockSpec((C, 2 * C), lambda b, i: (0, 0)),
            pl.BlockSpec((C, 1, 1), lambda b, i: (0, 0, 0)),
            pl.BlockSpec((C, C), lambda b, i: (0, 0)),
            pl.BlockSpec((C, C), lambda b, i: (0, 0)),
            pl.BlockSpec((C, 1, 1), lambda b, i: (0, 0, 0)),
            pl.BlockSpec((th4, H6), lambda b, i: (i, 0)),
            pl.BlockSpec((W6, W4), lambda b, i: (0, 0)),
        ],
        out_specs=[pl.BlockSpec((1, C, th4, W4), lambda b, i: (b, 0, i, 0))] * 3,
        compiler_params=_cparams(2),
    )(p4, p5, y6, w45, b45, wb34, wa56, b56, rh65, rw65t)

    th = H3 // 2
    p3_out, p3w = pl.pallas_call(
        _k3_body,
        out_shape=(jax.ShapeDtypeStruct((B, C, H3, W3), _F32),
                   jax.ShapeDtypeStruct((B, C, H3, W4), _BF)),
        grid=(B, 2),
        in_specs=[
            pl.BlockSpec((1, C, th, W3), lambda b, i: (b, 0, i, 0)),
            pl.BlockSpec((1, C, H4, W4), lambda b, i: (b, 0, 0, 0)),
            pl.BlockSpec((th, H4), lambda b, i: (i, 0)),
            pl.BlockSpec((W4, W3), lambda b, i: (0, 0)),
            pl.BlockSpec((C, C), lambda b, i: (0, 0)),
            pl.BlockSpec((C, 1, 1), lambda b, i: (0, 0, 0)),
            pl.BlockSpec((W3, W4), lambda b, i: (0, 0)),
        ],
        out_specs=[pl.BlockSpec((1, C, th, W3), lambda b, i: (b, 0, i, 0)),
                   pl.BlockSpec((1, C, th, W4), lambda b, i: (b, 0, i, 0))],
        compiler_params=_cparams(2),
    )(p3, y4, rh43, rw43t, wa34, b34, dw34t)

    p4_out = pl.pallas_call(
        _k4_body,
        out_shape=jax.ShapeDtypeStruct((B, C, H4, W4), _F32),
        grid=(B, 2),
        in_specs=[
            pl.BlockSpec((1, C, H3, W4), lambda b, i: (b, 0, 0, 0)),
            pl.BlockSpec((1, C, th4, W4), lambda b, i: (b, 0, i, 0)),
            pl.BlockSpec((1, C, th4, W4), lambda b, i: (b, 0, i, 0)),
            pl.BlockSpec((th4, H3), lambda b, i: (i, 0)),
            pl.BlockSpec((C, C), lambda b, i: (0, 0)),
            pl.BlockSpec((C, 1, 1), lambda b, i: (0, 0, 0)),
        ],
        out_specs=pl.BlockSpec((1, C, th4, W4), lambda b, i: (b, 0, i, 0)),
        compiler_params=_cparams(2),
    )(p3w, p4, p4_mid, dh34, w4e, b4e)

    p5_out, p6_out, p7_out = pl.pallas_call(
        _k5_body,
        out_shape=(jax.ShapeDtypeStruct((B, C, H4, W4), _F32),
                   jax.ShapeDtypeStruct((B, C, H6, W6), _F32),
                   jax.ShapeDtypeStruct((B, C, H6, W6), _F32)),
        grid=(B,),
        in_specs=[_full((1, C, H4, W4)), _full((1, C, H4, W4)),
                  _full((1, C, H4, W4)), _full((1, C, H6, W6)),
                  _full((1, C, H6, W6)), _full((1, C, H6, W6)),
                  cc((C, C)), cc((C, 1, 1)),
                  cc((H6, H4)), cc((W4, W6)),
                  cc((C, C)), cc((C, 1, 1)),
                  cc((C, C)), cc((C, 1, 1))],
        out_specs=[_full((1, C, H4, W4)), _full((1, C, H6, W6)),
                   _full((1, C, H6, W6))],
        compiler_params=_cparams(1),
    )(p4_out, p5, p5_mid, p6, p6_mid, p7,
      w5e, b5e, dh56, dw56t, w6e, b6e, w7e, b7e)

    return [p3_out, p4_out, p5_out, p6_out, p7_out]
